# baseline (speedup 1.0000x reference)
import functools

import jax
import jax.numpy as jnp
from jax import lax
from jax.experimental import pallas as pl
from jax.experimental.pallas import tpu as pltpu
from jax.experimental.pallas import tpu_sc as plsc

_N_IMGS = 100000
_BATCH = 16384
_NW = 32
_PER_W = _BATCH // _NW
_L = 16
_CSTRIDE = _N_IMGS // _L


def _build_gather_kernel():
    mesh = plsc.VectorSubcoreMesh(core_axis_name="c", subcore_axis_name="s")

    @functools.partial(
        pl.kernel,
        mesh=mesh,
        out_type=[
            jax.ShapeDtypeStruct((_BATCH * 4,), jnp.float32),
            jax.ShapeDtypeStruct((_BATCH * 3,), jnp.float32),
            jax.ShapeDtypeStruct((_L,), jnp.float32),
        ],
        scratch_types=[
            pltpu.VMEM((_PER_W,), jnp.int32),
            pltpu.VMEM((_PER_W,), jnp.int32),
            pltpu.VMEM((4 * _PER_W, _L), jnp.float32),
            pltpu.VMEM((3 * _PER_W, _L), jnp.float32),
            pltpu.VMEM((4 * _PER_W,), jnp.float32),
            pltpu.VMEM((3 * _PER_W,), jnp.float32),
            pltpu.VMEM((_L,), jnp.float32),
            pltpu.SemaphoreType.DMA,
            pltpu.SemaphoreType.DMA,
        ],
        compiler_params=pltpu.CompilerParams(
            use_tc_tiling_on_sc=False, needs_layout_passes=False,
            disable_bounds_checks=True, disable_semaphore_checks=True),
    )
    def gather_k(phi_hbm, t_hbm, f_hbm, idx_hbm,
                 phi_out, t_out, fx_out,
                 idx_v, gbase_v, phi_rows, t_rows, phi_x, t_x, f_v,
                 sem_a, sem_b):
        wid = lax.axis_index("s") * 2 + lax.axis_index("c")
        base = wid * _PER_W

        pltpu.sync_copy(idx_hbm.at[pl.ds(base, _PER_W)], idx_v)

        for s in range(_PER_W // _L):
            sl = pl.ds(s * _L, _L)
            gbase_v[sl] = lax.shift_right_logical(idx_v[sl], 4)

        copies = []
        for c in range(4):
            col = phi_hbm.at[pl.ds(c * _CSTRIDE, _CSTRIDE)]
            copies.append(pltpu.async_copy(
                col.at[gbase_v], phi_rows.at[pl.ds(c * _PER_W, _PER_W)],
                sem_a))
        for c in range(3):
            col = t_hbm.at[pl.ds(c * _CSTRIDE, _CSTRIDE)]
            copies.append(pltpu.async_copy(
                col.at[gbase_v], t_rows.at[pl.ds(c * _PER_W, _PER_W)],
                sem_b))
        for c in copies:
            c.wait()

        lanes = lax.iota(jnp.int32, _L)

        for s in range(_PER_W // _L):
            sl = pl.ds(s * _L, _L)
            col = lax.bitwise_and(idx_v[sl], _L - 1)
            row = s * _L + lanes
            for c in range(4):
                val = plsc.load_gather(phi_rows, [c * _PER_W + row, col])
                phi_x[pl.ds(c * _PER_W + s * _L, _L)] = val
            for c in range(3):
                val = plsc.load_gather(t_rows, [c * _PER_W + row, col])
                t_x[pl.ds(c * _PER_W + s * _L, _L)] = val

        for c in range(4):
            pltpu.sync_copy(phi_x.at[pl.ds(c * _PER_W, _PER_W)],
                            phi_out.at[pl.ds(c * _BATCH + base, _PER_W)])
        for c in range(3):
            pltpu.sync_copy(t_x.at[pl.ds(c * _PER_W, _PER_W)],
                            t_out.at[pl.ds(c * _BATCH + base, _PER_W)])

        @pl.when(wid == 0)
        def _():
            pltpu.sync_copy(f_hbm, f_v)
            val = f_v[...]
            f_v[...] = val * val * 1000.0
            pltpu.sync_copy(f_v, fx_out)

    return gather_k


_gather = _build_gather_kernel()


def kernel(phi, t, f, indices):
    idx = indices.astype(jnp.int32)
    phi_cm = phi.T.reshape(4 * _CSTRIDE, _L)
    t_cm = t.T.reshape(3 * _CSTRIDE, _L)
    f16 = jnp.broadcast_to(f.astype(jnp.float32), (_L,))
    phi_flat, t_flat, fx16 = _gather(phi_cm, t_cm, f16, idx)
    phi_sel = phi_flat.reshape(4, _BATCH).T
    t_sel = t_flat.reshape(3, _BATCH).T
    fx = fx16[:1]
    cx = jnp.asarray(500.0, jnp.float32)
    cy = jnp.asarray(500.0, jnp.float32)
    return (phi_sel, t_sel, fx, fx, cx, cy)


# speedup vs baseline: 1.0876x; 1.0876x over previous
import functools

import jax
import jax.numpy as jnp
from jax import lax
from jax.experimental import pallas as pl
from jax.experimental.pallas import tpu as pltpu
from jax.experimental.pallas import tpu_sc as plsc

_N_IMGS = 100000
_BATCH = 16384
_NW = 32
_PER_W = _BATCH // _NW
_L = 16
_CSTRIDE = _N_IMGS // _L


def _build_gather_kernel():
    mesh = plsc.VectorSubcoreMesh(core_axis_name="c", subcore_axis_name="s")

    @functools.partial(
        pl.kernel,
        mesh=mesh,
        out_type=[
            jax.ShapeDtypeStruct((_BATCH * 4,), jnp.float32),
            jax.ShapeDtypeStruct((_BATCH * 4,), jnp.float32),
            jax.ShapeDtypeStruct((_L,), jnp.float32),
        ],
        scratch_types=[
            pltpu.VMEM((_PER_W,), jnp.int32),
            pltpu.VMEM((4 * _PER_W,), jnp.int32),
            pltpu.VMEM((3 * _PER_W,), jnp.int32),
            pltpu.VMEM((4 * _PER_W, _L), jnp.float32),
            pltpu.VMEM((3 * _PER_W, _L), jnp.float32),
            pltpu.VMEM((4 * _PER_W,), jnp.float32),
            pltpu.VMEM((4 * _PER_W,), jnp.float32),
            pltpu.VMEM((_L,), jnp.float32),
            pltpu.SemaphoreType.DMA,
            pltpu.SemaphoreType.DMA,
        ],
        compiler_params=pltpu.CompilerParams(
            use_tc_tiling_on_sc=False, needs_layout_passes=False,
            disable_bounds_checks=True, disable_semaphore_checks=True),
    )
    def gather_k(phi_hbm, t_hbm, f_hbm, idx_hbm,
                 phi_out, t_out, fx_out,
                 idx_v, gphi_v, gt_v, phi_rows, t_rows, phi_x, t_x, f_v,
                 sem_a, sem_b):
        wid = lax.axis_index("s") * 2 + lax.axis_index("c")
        base = wid * _PER_W

        pltpu.sync_copy(idx_hbm.at[pl.ds(base, _PER_W)], idx_v)

        for s in range(_PER_W // _L):
            sl = pl.ds(s * _L, _L)
            g0 = lax.shift_right_logical(idx_v[sl], 4)
            for c in range(4):
                gphi_v[pl.ds(c * _PER_W + s * _L, _L)] = g0 + (c * _CSTRIDE)
            for c in range(3):
                gt_v[pl.ds(c * _PER_W + s * _L, _L)] = g0 + (c * _CSTRIDE)

        cp_phi = pltpu.async_copy(phi_hbm.at[gphi_v], phi_rows, sem_a)
        cp_t = pltpu.async_copy(t_hbm.at[gt_v], t_rows, sem_b)

        lanes = lax.iota(jnp.int32, _L)
        cp_phi.wait()
        for s in range(_PER_W // _L):
            sl = pl.ds(s * _L, _L)
            col = lax.bitwise_and(idx_v[sl], _L - 1)
            row = s * _L + lanes
            dst = (s >> 3) * 512 + (s & 7) * _L
            for c in range(4):
                val = plsc.load_gather(phi_rows, [c * _PER_W + row, col])
                phi_x[pl.ds(dst + c * 128, _L)] = val
        pltpu.sync_copy(phi_x, phi_out.at[pl.ds(base * 4, 4 * _PER_W)])

        cp_t.wait()
        for s in range(_PER_W // _L):
            sl = pl.ds(s * _L, _L)
            col = lax.bitwise_and(idx_v[sl], _L - 1)
            row = s * _L + lanes
            dst = (s >> 3) * 512 + (s & 7) * _L
            for c in range(3):
                val = plsc.load_gather(t_rows, [c * _PER_W + row, col])
                t_x[pl.ds(dst + c * 128, _L)] = val
        pltpu.sync_copy(t_x, t_out.at[pl.ds(base * 4, 4 * _PER_W)])

        @pl.when(wid == 0)
        def _():
            pltpu.sync_copy(f_hbm, f_v)
            val = f_v[...]
            f_v[...] = val * val * 1000.0
            pltpu.sync_copy(f_v, fx_out)

    return gather_k


_gather = _build_gather_kernel()


def kernel(phi, t, f, indices):
    idx = indices.astype(jnp.int32)
    phi_cm = phi.T.reshape(4 * _CSTRIDE, _L)
    t_cm = t.T.reshape(3 * _CSTRIDE, _L)
    f16 = jnp.broadcast_to(f.astype(jnp.float32), (_L,))
    phi_sw, t_flat, fx16 = _gather(phi_cm, t_cm, f16, idx)
    phi_sel = (phi_sw.reshape(_BATCH // 128, 4, 128)
               .transpose(0, 2, 1).reshape(_BATCH, 4))
    t_sel = (t_flat.reshape(_BATCH // 128, 4, 128)
             .transpose(0, 2, 1).reshape(_BATCH, 4))[:, :3]
    fx = fx16[:1]
    cx = jnp.asarray(500.0, jnp.float32)
    cy = jnp.asarray(500.0, jnp.float32)
    return (phi_sel, t_sel, fx, fx, cx, cy)


# speedup vs baseline: 1.1175x; 1.0275x over previous
import functools

import jax
import jax.numpy as jnp
from jax import lax
from jax.experimental import pallas as pl
from jax.experimental.pallas import tpu as pltpu
from jax.experimental.pallas import tpu_sc as plsc

_N_IMGS = 100000
_BATCH = 16384
_NW = 32
_PER_W = _BATCH // _NW
_L = 16
_CSTRIDE = _N_IMGS // _L


def _build_gather_kernel():
    mesh = plsc.VectorSubcoreMesh(core_axis_name="c", subcore_axis_name="s")

    @functools.partial(
        pl.kernel,
        mesh=mesh,
        out_type=[
            jax.ShapeDtypeStruct((_BATCH * 4,), jnp.float32),
            jax.ShapeDtypeStruct((_BATCH * 4,), jnp.float32),
            jax.ShapeDtypeStruct((_L,), jnp.float32),
        ],
        scratch_types=[
            pltpu.VMEM((_PER_W,), jnp.int32),
            pltpu.VMEM((4 * _PER_W,), jnp.int32),
            pltpu.VMEM((3 * _PER_W,), jnp.int32),
            pltpu.VMEM((4 * _PER_W, _L), jnp.float32),
            pltpu.VMEM((3 * _PER_W, _L), jnp.float32),
            pltpu.VMEM((4 * _PER_W,), jnp.float32),
            pltpu.VMEM((4 * _PER_W,), jnp.float32),
            pltpu.VMEM((_L,), jnp.float32),
            pltpu.SemaphoreType.DMA,
            pltpu.SemaphoreType.DMA,
        ],
        compiler_params=pltpu.CompilerParams(
            use_tc_tiling_on_sc=False, needs_layout_passes=False,
            disable_bounds_checks=True, disable_semaphore_checks=True),
    )
    def gather_k(phi_hbm, t_hbm, f_hbm, idx_hbm,
                 phi_out, t_out, fx_out,
                 idx_v, gphi_v, gt_v, phi_rows, t_rows, phi_x, t_x,
                 f_v, sem_a, sem_b):
        wid = lax.axis_index("s") * 2 + lax.axis_index("c")
        base = wid * _PER_W

        pltpu.sync_copy(idx_hbm.at[pl.ds(base, _PER_W)], idx_v)

        lanes = lax.iota(jnp.int32, _L)

        def gidx_body(s, carry):
            p = s * _L + lanes
            g0 = lax.shift_right_logical(plsc.load_gather(idx_v, [p]), 4)
            for c in range(4):
                plsc.store_scatter(gphi_v, [c * _PER_W + p], g0 + c * _CSTRIDE)
            for c in range(3):
                plsc.store_scatter(gt_v, [c * _PER_W + p], g0 + c * _CSTRIDE)
            return carry

        lax.fori_loop(0, _PER_W // _L, gidx_body, 0)

        cp_phi = pltpu.async_copy(phi_hbm.at[gphi_v], phi_rows, sem_a)
        cp_t = pltpu.async_copy(t_hbm.at[gt_v], t_rows, sem_b)

        @pl.when(wid == 0)
        def _():
            pltpu.sync_copy(f_hbm, f_v)
            val = f_v[...]
            f_v[...] = val * val * 1000.0
            pltpu.sync_copy(f_v, fx_out)

        cp_phi.wait()

        def phi_body(s, carry):
            p = s * _L + lanes
            iv = plsc.load_gather(idx_v, [p])
            col = lax.bitwise_and(iv, _L - 1)
            dst = (lax.shift_right_logical(s, 3) * 512
                   + lax.bitwise_and(s, 7) * _L + lanes)
            for c in range(4):
                val = plsc.load_gather(phi_rows, [c * _PER_W + p, col])
                plsc.store_scatter(phi_x, [dst + c * 128], val)
            return carry

        lax.fori_loop(0, _PER_W // _L, phi_body, 0)
        pltpu.sync_copy(phi_x, phi_out.at[pl.ds(base * 4, 4 * _PER_W)])

        cp_t.wait()

        def t_body(s, carry):
            p = s * _L + lanes
            iv = plsc.load_gather(idx_v, [p])
            col = lax.bitwise_and(iv, _L - 1)
            dst = (lax.shift_right_logical(s, 3) * 512
                   + lax.bitwise_and(s, 7) * _L + lanes)
            for c in range(3):
                val = plsc.load_gather(t_rows, [c * _PER_W + p, col])
                plsc.store_scatter(t_x, [dst + c * 128], val)
            return carry

        lax.fori_loop(0, _PER_W // _L, t_body, 0)
        pltpu.sync_copy(t_x, t_out.at[pl.ds(base * 4, 4 * _PER_W)])

    return gather_k


_gather = _build_gather_kernel()


def kernel(phi, t, f, indices):
    idx = indices.astype(jnp.int32)
    phi_cm = phi.T.reshape(4 * _CSTRIDE, _L)
    t_cm = t.T.reshape(3 * _CSTRIDE, _L)
    f16 = jnp.broadcast_to(f.astype(jnp.float32), (_L,))
    phi_sw, t_flat, fx16 = _gather(phi_cm, t_cm, f16, idx)
    phi_sel = (phi_sw.reshape(_BATCH // 128, 4, 128)
               .transpose(0, 2, 1).reshape(_BATCH, 4))
    t_sel = (t_flat.reshape(_BATCH // 128, 4, 128)
             .transpose(0, 2, 1).reshape(_BATCH, 4))[:, :3]
    fx = fx16[:1]
    cx = jnp.asarray(500.0, jnp.float32)
    cy = jnp.asarray(500.0, jnp.float32)
    return (phi_sel, t_sel, fx, fx, cx, cy)


# speedup vs baseline: 1.1198x; 1.0021x over previous
import functools

import jax
import jax.numpy as jnp
from jax import lax
from jax.experimental import pallas as pl
from jax.experimental.pallas import tpu as pltpu
from jax.experimental.pallas import tpu_sc as plsc

_N_IMGS = 100000
_BATCH = 16384
_NW = 32
_PER_W = _BATCH // _NW
_L = 16
_W = 8
_CSTRIDE = _N_IMGS // _W


def _build_gather_kernel():
    mesh = plsc.VectorSubcoreMesh(core_axis_name="c", subcore_axis_name="s")

    @functools.partial(
        pl.kernel,
        mesh=mesh,
        out_type=[
            jax.ShapeDtypeStruct((_BATCH * 4,), jnp.float32),
            jax.ShapeDtypeStruct((_BATCH * 4,), jnp.float32),
            jax.ShapeDtypeStruct((_L,), jnp.float32),
        ],
        scratch_types=[
            pltpu.VMEM((_PER_W,), jnp.int32),
            pltpu.VMEM((4 * _PER_W,), jnp.int32),
            pltpu.VMEM((3 * _PER_W,), jnp.int32),
            pltpu.VMEM((4 * _PER_W, _W), jnp.float32),
            pltpu.VMEM((3 * _PER_W, _W), jnp.float32),
            pltpu.VMEM((4 * _PER_W,), jnp.float32),
            pltpu.VMEM((4 * _PER_W,), jnp.float32),
            pltpu.VMEM((_L,), jnp.float32),
            pltpu.SemaphoreType.DMA,
            pltpu.SemaphoreType.DMA,
        ],
        compiler_params=pltpu.CompilerParams(
            use_tc_tiling_on_sc=False, needs_layout_passes=False,
            disable_bounds_checks=True, disable_semaphore_checks=True),
    )
    def gather_k(phi_hbm, t_hbm, f_hbm, idx_hbm,
                 phi_out, t_out, fx_out,
                 idx_v, gphi_v, gt_v, phi_rows, t_rows, phi_x, t_x,
                 f_v, sem_a, sem_b):
        wid = lax.axis_index("s") * 2 + lax.axis_index("c")
        base = wid * _PER_W

        pltpu.sync_copy(idx_hbm.at[pl.ds(base, _PER_W)], idx_v)

        lanes = lax.iota(jnp.int32, _L)

        def gidx_body(s, carry):
            p = s * _L + lanes
            g0 = lax.shift_right_logical(plsc.load_gather(idx_v, [p]), 3)
            for c in range(4):
                plsc.store_scatter(gphi_v, [c * _PER_W + p], g0 + c * _CSTRIDE)
            for c in range(3):
                plsc.store_scatter(gt_v, [c * _PER_W + p], g0 + c * _CSTRIDE)
            return carry

        lax.fori_loop(0, _PER_W // _L, gidx_body, 0)

        cp_phi = pltpu.async_copy(phi_hbm.at[gphi_v], phi_rows, sem_a)
        cp_t = pltpu.async_copy(t_hbm.at[gt_v], t_rows, sem_b)

        @pl.when(wid == 0)
        def _():
            pltpu.sync_copy(f_hbm, f_v)
            val = f_v[...]
            f_v[...] = val * val * 1000.0
            pltpu.sync_copy(f_v, fx_out)

        cp_phi.wait()

        def phi_body(s, carry):
            p = s * _L + lanes
            iv = plsc.load_gather(idx_v, [p])
            col = lax.bitwise_and(iv, _W - 1)
            dst = (lax.shift_right_logical(s, 3) * 512
                   + lax.bitwise_and(s, 7) * _L + lanes)
            for c in range(4):
                val = plsc.load_gather(phi_rows, [c * _PER_W + p, col])
                plsc.store_scatter(phi_x, [dst + c * 128], val)
            return carry

        lax.fori_loop(0, _PER_W // _L, phi_body, 0)
        pltpu.sync_copy(phi_x, phi_out.at[pl.ds(base * 4, 4 * _PER_W)])

        cp_t.wait()

        def t_body(s, carry):
            p = s * _L + lanes
            iv = plsc.load_gather(idx_v, [p])
            col = lax.bitwise_and(iv, _W - 1)
            dst = (lax.shift_right_logical(s, 3) * 512
                   + lax.bitwise_and(s, 7) * _L + lanes)
            for c in range(3):
                val = plsc.load_gather(t_rows, [c * _PER_W + p, col])
                plsc.store_scatter(t_x, [dst + c * 128], val)
            return carry

        lax.fori_loop(0, _PER_W // _L, t_body, 0)
        pltpu.sync_copy(t_x, t_out.at[pl.ds(base * 4, 4 * _PER_W)])

    return gather_k


_gather = _build_gather_kernel()


def kernel(phi, t, f, indices):
    idx = indices.astype(jnp.int32)
    phi_cm = phi.T.reshape(4 * _CSTRIDE, _W)
    t_cm = t.T.reshape(3 * _CSTRIDE, _W)
    f16 = jnp.broadcast_to(f.astype(jnp.float32), (_L,))
    phi_sw, t_flat, fx16 = _gather(phi_cm, t_cm, f16, idx)
    phi_sel = (phi_sw.reshape(_BATCH // 128, 4, 128)
               .transpose(0, 2, 1).reshape(_BATCH, 4))
    t_sel = (t_flat.reshape(_BATCH // 128, 4, 128)
             .transpose(0, 2, 1).reshape(_BATCH, 4))[:, :3]
    fx = fx16[:1]
    cx = jnp.asarray(500.0, jnp.float32)
    cy = jnp.asarray(500.0, jnp.float32)
    return (phi_sel, t_sel, fx, fx, cx, cy)
